# Initial kernel scaffold; baseline (speedup 1.0000x reference)
import jax
import jax.numpy as jnp
from jax.experimental import pallas as pl


def kernel(codon_indices, codon_table):
    raise NotImplementedError("write your pallas kernel here")


# speedup vs baseline: 97.7180x; 97.7180x over previous
import functools

import jax
import jax.numpy as jnp
from jax import lax
from jax.experimental import pallas as pl
from jax.experimental.pallas import tpu as pltpu
from jax.experimental.pallas import tpu_sc as plsc

_VOCAB = 64
_LATENT = 64
_BATCH = 512
_SEQ = 2048
_NC = 2
_NS = 16
_L = 16
_NW = _NC * _NS
_ROWS = _BATCH // _NW
_CHUNKS = _SEQ // _L
_KL = _LATENT // _L


def _build_sc_kernel():
    mesh = plsc.VectorSubcoreMesh(core_axis_name="c", subcore_axis_name="s")

    @functools.partial(
        pl.kernel,
        mesh=mesh,
        out_type=jax.ShapeDtypeStruct((_BATCH, _LATENT), jnp.float32),
        compiler_params=pltpu.CompilerParams(needs_layout_passes=False),
        scratch_types=[
            pltpu.VMEM((_ROWS, _SEQ), jnp.int32),
            pltpu.VMEM((_VOCAB, _LATENT), jnp.float32),
            pltpu.VMEM((_VOCAB,), jnp.float32),
            pltpu.VMEM((_ROWS, _LATENT), jnp.float32),
        ],
    )
    def k(idx_hbm, table_hbm, out_hbm, idx_v, table_v, counts_v, out_v):
        wid = lax.axis_index("s") * _NC + lax.axis_index("c")
        base = wid * _ROWS
        pltpu.sync_copy(idx_hbm.at[pl.ds(base, _ROWS)], idx_v)
        pltpu.sync_copy(table_hbm, table_v)

        ones = jnp.ones((_L,), jnp.float32)
        zeros = jnp.zeros((_L,), jnp.float32)
        inv = jnp.full((_L,), 1.0 / _SEQ, jnp.float32)

        def row_body(r, carry):
            for j in range(_VOCAB // _L):
                counts_v[pl.ds(j * _L, _L)] = zeros

            def hist_body(i, c):
                vec = idx_v[r, pl.ds(i * _L, _L)]
                plsc.addupdate_scatter(counts_v, [vec], ones)
                return c

            lax.fori_loop(0, _CHUNKS, hist_body, 0)

            def mm_body(v, accs):
                cvec = plsc.load_gather(counts_v, [jnp.full((_L,), v, jnp.int32)])
                return tuple(
                    accs[q] + cvec * table_v[v, pl.ds(q * _L, _L)]
                    for q in range(_KL)
                )

            accs = lax.fori_loop(
                0, _VOCAB, mm_body,
                tuple(jnp.zeros((_L,), jnp.float32) for _ in range(_KL)),
            )
            for q in range(_KL):
                out_v[r, pl.ds(q * _L, _L)] = accs[q] * inv
            return carry

        lax.fori_loop(0, _ROWS, row_body, 0)
        pltpu.sync_copy(out_v, out_hbm.at[pl.ds(base, _ROWS)])

    return k


_SC_KERNEL = _build_sc_kernel()


def kernel(codon_indices, codon_table):
    return _SC_KERNEL(codon_indices.astype(jnp.int32),
                      codon_table.astype(jnp.float32))
